# baseline (speedup 1.0000x reference)
import functools

import jax
import jax.numpy as jnp
from jax import lax
from jax.experimental import pallas as pl
from jax.experimental.pallas import tpu as pltpu
from jax.experimental.pallas import tpu_sc as plsc

NC = 2
NS = 16
LANES = 16

EMBED_DIM = 128
CHUNK = 100


def _embed_kernel(n_chunks_per_w, x_hbm, tok_hbm, pos_hbm, out_hbm,
                  idx_v, pos_v, gbuf, sbuf, gsem, ssem, psem):
    wid = lax.axis_index("s") * NC + lax.axis_index("c")
    row0 = wid * n_chunks_per_w

    pltpu.sync_copy(x_hbm.at[pl.ds(row0, n_chunks_per_w)], idx_v)
    pos_cp = pltpu.async_copy(pos_hbm, pos_v, psem)

    n_sub = EMBED_DIM // LANES

    def fire_gather(c, b):
        pltpu.async_copy(tok_hbm.at[idx_v.at[c]], gbuf.at[b], gsem[b])

    fire_gather(0, 0)
    fire_gather(1, 1)
    pos_cp.wait()

    def step(g, carry):
        for b in range(2):
            c = g * 2 + b
            pltpu.make_async_copy(tok_hbm.at[idx_v.at[c]],
                                  gbuf.at[b], gsem[b]).wait()
            @pl.when(g > 0)
            def _():
                pltpu.make_async_copy(
                    sbuf.at[b],
                    out_hbm.at[pl.ds((row0 + c - 2) * CHUNK, CHUNK)],
                    ssem[b]).wait()

            po = lax.rem(c, 2) * CHUNK

            def add_body(r, carry2):
                for d in range(n_sub):
                    sl = pl.ds(d * LANES, LANES)
                    sbuf[b, r, sl] = gbuf[b, r, sl] + pos_v[po + r, sl]
                return carry2

            lax.fori_loop(0, CHUNK, add_body, 0, unroll=4)

            pltpu.async_copy(sbuf.at[b],
                             out_hbm.at[pl.ds((row0 + c) * CHUNK, CHUNK)],
                             ssem[b])

            @pl.when(c + 2 < n_chunks_per_w)
            def _():
                fire_gather(c + 2, b)
        return carry

    lax.fori_loop(0, n_chunks_per_w // 2, step, 0)

    for b in range(2):
        c = n_chunks_per_w - 2 + b
        pltpu.make_async_copy(sbuf.at[b],
                              out_hbm.at[pl.ds((row0 + c) * CHUNK, CHUNK)],
                              ssem[b]).wait()


def kernel(x, token_table, pos_table):
    B, S = x.shape
    D = token_table.shape[1]
    n_lookups = B * S
    n_w = NC * NS
    n_chunks = n_lookups // CHUNK
    n_chunks_per_w = n_chunks // n_w

    x_rows = x.reshape(n_chunks, CHUNK).astype(jnp.int32)

    mesh = plsc.VectorSubcoreMesh(
        core_axis_name="c", subcore_axis_name="s",
        num_cores=NC, num_subcores=NS)

    out_flat = pl.kernel(
        functools.partial(_embed_kernel, n_chunks_per_w),
        out_type=jax.ShapeDtypeStruct((n_lookups, D), jnp.float32),
        mesh=mesh,
        scratch_types=[
            pltpu.VMEM((n_chunks_per_w, CHUNK), jnp.int32),
            pltpu.VMEM((S, D), jnp.float32),
            pltpu.VMEM((2, CHUNK, D), jnp.float32),
            pltpu.VMEM((2, CHUNK, D), jnp.float32),
            [pltpu.SemaphoreType.DMA, pltpu.SemaphoreType.DMA],
            [pltpu.SemaphoreType.DMA, pltpu.SemaphoreType.DMA],
            pltpu.SemaphoreType.DMA,
        ],
        compiler_params=pltpu.CompilerParams(use_tc_tiling_on_sc=False),
    )(x_rows, token_table, pos_table)

    return out_flat.reshape(B, S, D)


# speedup vs baseline: 2.6270x; 2.6270x over previous
import functools

import jax
import jax.numpy as jnp
from jax import lax
from jax.experimental import pallas as pl
from jax.experimental.pallas import tpu as pltpu
from jax.experimental.pallas import tpu_sc as plsc

NC = 2
NS = 16
LANES = 16

EMBED_DIM = 128
CHUNK = 100


def _embed_kernel(n_chunks_per_w, x_hbm, tok_hbm, pos_hbm, out_hbm,
                  idx_v, pos_v, gbuf, sbuf, gsem, ssem, psem):
    wid = lax.axis_index("s") * NC + lax.axis_index("c")
    row0 = wid * n_chunks_per_w

    pltpu.sync_copy(x_hbm.at[pl.ds(row0, n_chunks_per_w)], idx_v)
    pos_cp = pltpu.async_copy(pos_hbm, pos_v, psem)

    n_sub = EMBED_DIM // LANES

    def fire_gather(c, b):
        pltpu.async_copy(tok_hbm.at[idx_v.at[c]], gbuf.at[b], gsem[b])

    fire_gather(0, 0)
    fire_gather(1, 1)
    pos_cp.wait()

    def step(g, carry):
        for b in range(2):
            c = g * 2 + b
            pltpu.make_async_copy(tok_hbm.at[idx_v.at[c]],
                                  gbuf.at[b], gsem[b]).wait()
            @pl.when(g > 0)
            def _():
                pltpu.make_async_copy(
                    sbuf.at[b],
                    out_hbm.at[pl.ds((row0 + c - 2) * CHUNK, CHUNK)],
                    ssem[b]).wait()

            po = lax.rem(c, 2) * CHUNK

            def add_body(r, carry2):
                for d in range(n_sub):
                    sl = pl.ds(d * LANES, LANES)
                    sbuf[b, r, sl] = gbuf[b, r, sl] + pos_v[po + r, sl]
                return carry2

            lax.fori_loop(0, CHUNK, add_body, 0)

            pltpu.async_copy(sbuf.at[b],
                             out_hbm.at[pl.ds((row0 + c) * CHUNK, CHUNK)],
                             ssem[b])

            @pl.when(c + 2 < n_chunks_per_w)
            def _():
                fire_gather(c + 2, b)
        return carry

    lax.fori_loop(0, n_chunks_per_w // 2, step, 0)

    for b in range(2):
        c = n_chunks_per_w - 2 + b
        pltpu.make_async_copy(sbuf.at[b],
                              out_hbm.at[pl.ds((row0 + c) * CHUNK, CHUNK)],
                              ssem[b]).wait()


def kernel(x, token_table, pos_table):
    B, S = x.shape
    D = token_table.shape[1]
    n_lookups = B * S
    n_w = NC * NS
    n_chunks = n_lookups // CHUNK
    n_chunks_per_w = n_chunks // n_w

    x_rows = x.reshape(n_chunks, CHUNK).astype(jnp.int32)

    mesh = plsc.VectorSubcoreMesh(
        core_axis_name="c", subcore_axis_name="s",
        num_cores=NC, num_subcores=NS)

    out_flat = pl.kernel(
        functools.partial(_embed_kernel, n_chunks_per_w),
        out_type=jax.ShapeDtypeStruct((n_lookups, D), jnp.float32),
        mesh=mesh,
        scratch_types=[
            pltpu.VMEM((n_chunks_per_w, CHUNK), jnp.int32),
            pltpu.VMEM((S, D), jnp.float32),
            pltpu.VMEM((2, CHUNK, D), jnp.float32),
            pltpu.VMEM((2, CHUNK, D), jnp.float32),
            [pltpu.SemaphoreType.DMA, pltpu.SemaphoreType.DMA],
            [pltpu.SemaphoreType.DMA, pltpu.SemaphoreType.DMA],
            pltpu.SemaphoreType.DMA,
        ],
        compiler_params=pltpu.CompilerParams(use_tc_tiling_on_sc=False),
    )(x_rows, token_table, pos_table)

    return out_flat.reshape(B, S, D)
